# baseline (speedup 1.0000x reference)
import functools

import jax
import jax.numpy as jnp
from jax import lax
from jax.experimental import pallas as pl
from jax.experimental.pallas import tpu as pltpu

B = 128
N = 100000
BR = 8
GRID = B // BR

_KS0 = 0
_KS1 = 42
_KS2 = _KS0 ^ _KS1 ^ 0x1BD11BDA

_ROT_A = (13, 15, 26, 6)
_ROT_B = (17, 29, 16, 24)


def _threefry2x32(x0, x1):
    ks = (jnp.uint32(_KS0), jnp.uint32(_KS1), jnp.uint32(_KS2))
    x0 = x0 + ks[0]
    x1 = x1 + ks[1]
    for g in range(5):
        rots = _ROT_A if g % 2 == 0 else _ROT_B
        for r in rots:
            x0 = x0 + x1
            x1 = (x1 << r) | (x1 >> (32 - r))
            x1 = x1 ^ x0
        x0 = x0 + ks[(g + 1) % 3]
        x1 = x1 + ks[(g + 2) % 3] + jnp.uint32(g + 1)
    return x0 ^ x1


def _body(x_ref, probs_ref, y_ref):
    xb = x_ref[...]
    m = jnp.max(xb, axis=1, keepdims=True)
    e = jnp.exp(xb - m)
    s = jnp.sum(e, axis=1, keepdims=True)
    probs_ref[...] = e / s

    pid = pl.program_id(0)
    row = lax.broadcasted_iota(jnp.uint32, (BR, N), 0)
    col = lax.broadcasted_iota(jnp.uint32, (BR, N), 1)
    idx = (jnp.uint32(pid) * jnp.uint32(BR) + row) * jnp.uint32(N) + col

    bits = _threefry2x32(jnp.zeros_like(idx), idx)
    fb = (bits >> 9) | jnp.uint32(0x3F800000)
    u = lax.bitcast_convert_type(fb, jnp.float32) - jnp.float32(1.0)
    tiny = jnp.float32(jnp.finfo(jnp.float32).tiny)
    u = jnp.maximum(tiny, u * (jnp.float32(1.0) - tiny) + tiny)
    g = -jnp.log(-jnp.log(u))

    val = xb + g
    vmax = jnp.max(val, axis=1, keepdims=True)
    big = jnp.int32(0x7FFFFFFF)
    cand = jnp.where(val == vmax, col.astype(jnp.int32), big)
    y_ref[0, 0, :] = jnp.min(cand, axis=1)


@jax.jit
def kernel(x):
    probs, y3 = pl.pallas_call(
        _body,
        grid=(GRID,),
        in_specs=[pl.BlockSpec((BR, N), lambda i: (i, 0))],
        out_specs=[
            pl.BlockSpec((BR, N), lambda i: (i, 0)),
            pl.BlockSpec((1, 1, BR), lambda i: (i, 0, 0)),
        ],
        out_shape=[
            jax.ShapeDtypeStruct((B, N), jnp.float32),
            jax.ShapeDtypeStruct((GRID, 1, BR), jnp.int32),
        ],
    )(x)
    return (y3.reshape(B), probs)
